# baseline (speedup 1.0000x reference)
import functools
import jax
import jax.numpy as jnp
from jax import lax
from jax.experimental import pallas as pl
from jax.experimental.pallas import tpu as pltpu
from jax.experimental.pallas import tpu_sc as plsc

N = 10000
E = 320000
D = 128
NC = 2
NS = 16
NW = NC * NS
CHUNK = 32
CPW = 320
NCHUNK = CPW * NW
EPAD = NCHUNK * CHUNK
GC = 64
NB = 4
SEGV = 80
SEGP = 40
NPAD = 10240
RPS = NPAD // NS
DEGW = 16
RB = 1024

_mesh = plsc.VectorSubcoreMesh(
    core_axis_name="c", subcore_axis_name="s", num_cores=NC, num_subcores=NS)


@functools.partial(
    pl.kernel,
    out_type=jax.ShapeDtypeStruct((NC, NPAD, DEGW), jnp.float32),
    mesh=_mesh,
    scratch_types=[
        pltpu.VMEM((CPW // 4, 4 * CHUNK), jnp.int32),
        pltpu.VMEM((4 * CHUNK, DEGW), jnp.float32),
        pltpu.VMEM((16, DEGW), jnp.float32),
        pltpu.VMEM_SHARED((NPAD, DEGW), jnp.float32),
    ],
)
def _deg_kernel(row_hbm, out_hbm, idx_v, ones_v, zb_v, deg_sh):
    cid = lax.axis_index("c")
    sid = lax.axis_index("s")
    wid = sid * NC + cid

    def fill_ones(i, _):
        ones_v[i, :] = jnp.ones((DEGW,), jnp.float32)
        return 0
    lax.fori_loop(0, 4 * CHUNK, fill_ones, 0)

    def fill_z(i, _):
        zb_v[i, :] = jnp.zeros((DEGW,), jnp.float32)
        return 0
    lax.fori_loop(0, 16, fill_z, 0)

    def zero_acc(i, _):
        pltpu.sync_copy(zb_v, deg_sh.at[pl.ds(sid * RPS + i * 16, 16)])
        return 0
    lax.fori_loop(0, RPS // 16, zero_acc, 0)
    plsc.subcore_barrier()

    pltpu.sync_copy(row_hbm.at[pl.ds(wid * (CPW // 4), CPW // 4)], idx_v)

    def body(j, _):
        pltpu.sync_copy(ones_v, deg_sh.at[idx_v.at[j]], add=True)
        return 0
    lax.fori_loop(0, CPW // 4, body, 0)
    plsc.subcore_barrier()

    pltpu.sync_copy(deg_sh.at[pl.ds(sid * RPS, RPS)],
                    out_hbm.at[cid, pl.ds(sid * RPS, RPS)])


@functools.partial(
    pl.kernel,
    out_type=jax.ShapeDtypeStruct((NC, NPAD, D), jnp.float32),
    mesh=_mesh,
    scratch_types=[
        pltpu.VMEM((SEGP, 4 * CHUNK), jnp.int32),
        pltpu.VMEM((SEGP, 4 * CHUNK), jnp.int32),
        pltpu.VMEM((NB, GC, D), jnp.float32),
        pltpu.VMEM((16, D), jnp.float32),
        pltpu.VMEM_SHARED((NPAD, D), jnp.float32),
        [pltpu.SemaphoreType.DMA] * NB,
        [pltpu.SemaphoreType.DMA] * NB,
    ],
)
def _edge_scatter(hp_hbm, row_hbm, col_hbm, out_hbm,
                  idxr_v, idxc_v, rows_v, zb_v, acc_sh, gsems, ssems):
    cid = lax.axis_index("c")
    sid = lax.axis_index("s")
    wid = sid * NC + cid

    def fill_z(i, _):
        def fz(k, _):
            zb_v[i, pl.ds(k * 16, 16)] = jnp.zeros((16,), jnp.float32)
            return 0
        lax.fori_loop(0, D // 16, fz, 0)
        return 0
    lax.fori_loop(0, 16, fill_z, 0)

    def zero_acc(i, _):
        pltpu.sync_copy(zb_v, acc_sh.at[pl.ds(sid * RPS + i * 16, 16)])
        return 0
    lax.fori_loop(0, RPS // 16, zero_acc, 0)
    plsc.subcore_barrier()

    GD = NB - 2
    for h in range(2):
        base = wid * (CPW // 4) + h * SEGP
        pltpu.sync_copy(row_hbm.at[pl.ds(base, SEGP)], idxr_v)
        pltpu.sync_copy(col_hbm.at[pl.ds(base, SEGP)], idxc_v)

        def idx_of(v, j):
            return v.at[j // 2, pl.ds((j % 2) * GC, GC)]

        def wait_g(b):
            pltpu.make_async_copy(
                hp_hbm.at[idx_of(idxr_v, 0)], rows_v.at[b], gsems[b]).wait()

        def start_g(b, j):
            pltpu.async_copy(
                hp_hbm.at[idx_of(idxr_v, j)], rows_v.at[b], gsems[b])

        def wait_s(b):
            pltpu.make_async_copy(
                rows_v.at[b], acc_sh.at[idx_of(idxc_v, 0)], ssems[b]).wait()

        def start_s(b, j):
            pltpu.async_copy(
                rows_v.at[b], acc_sh.at[idx_of(idxc_v, j)], ssems[b],
                add=True)

        for b in range(GD):
            start_g(b, b)

        for j in range(NB):
            b = j % NB
            wait_g(b)
            start_s(b, j)
            jd = j + GD
            if jd - NB >= 0:
                wait_s(jd % NB)
            start_g(jd % NB, jd)

        def group(g, _):
            for b in range(NB):
                j = g * NB + b
                wait_g(b)
                start_s(b, j)
                wait_s((b + GD) % NB)
                start_g((b + GD) % NB, j + GD)
            return 0
        lax.fori_loop(1, SEGV // NB - 1, group, 0)

        for j in range(SEGV - NB, SEGV):
            b = j % NB
            wait_g(b)
            start_s(b, j)
            jd = j + GD
            if jd < SEGV:
                wait_s(jd % NB)
                start_g(jd % NB, jd)

        for b in range(NB):
            wait_s(b)
    plsc.subcore_barrier()

    pltpu.sync_copy(acc_sh.at[pl.ds(sid * RPS, RPS)],
                    out_hbm.at[cid, pl.ds(sid * RPS, RPS)])


def _first_body(deg_ref, x_ref, w_ref, b_ref, hp_ref, dinv_ref):
    deg = deg_ref[0, :, 0:1] + deg_ref[1, :, 0:1] + 1.0
    dinv = lax.rsqrt(deg)
    z = jnp.dot(x_ref[...], w_ref[...],
                preferred_element_type=jnp.float32) + b_ref[...]
    hp_ref[...] = dinv * z
    dinv_ref[...] = dinv


def _first_call(degp, xp, W1t, b1r):
    return pl.pallas_call(
        _first_body,
        grid=(NPAD // RB,),
        in_specs=[
            pl.BlockSpec((2, RB, DEGW), lambda i: (0, i, 0)),
            pl.BlockSpec((RB, D), lambda i: (i, 0)),
            pl.BlockSpec((D, D), lambda i: (0, 0)),
            pl.BlockSpec((1, D), lambda i: (0, 0)),
        ],
        out_specs=[
            pl.BlockSpec((RB, D), lambda i: (i, 0)),
            pl.BlockSpec((RB, 1), lambda i: (i, 0)),
        ],
        out_shape=[
            jax.ShapeDtypeStruct((NPAD, D), jnp.float32),
            jax.ShapeDtypeStruct((NPAD, 1), jnp.float32),
        ],
    )(degp, xp, W1t, b1r)


def _mid_body(p_ref, hp_ref, dinv_ref, w_ref, b_ref, hp2_ref):
    dinv = dinv_ref[...]
    h1 = jnp.tanh(dinv * (p_ref[0] + p_ref[1] + hp_ref[...]))
    z = jnp.dot(h1, w_ref[...], preferred_element_type=jnp.float32) + b_ref[...]
    hp2_ref[...] = dinv * z


def _mid_call(P, hp1, dinv, W2t, b2r):
    return pl.pallas_call(
        _mid_body,
        grid=(NPAD // RB,),
        in_specs=[
            pl.BlockSpec((2, RB, D), lambda i: (0, i, 0)),
            pl.BlockSpec((RB, D), lambda i: (i, 0)),
            pl.BlockSpec((RB, 1), lambda i: (i, 0)),
            pl.BlockSpec((D, D), lambda i: (0, 0)),
            pl.BlockSpec((1, D), lambda i: (0, 0)),
        ],
        out_specs=pl.BlockSpec((RB, D), lambda i: (i, 0)),
        out_shape=jax.ShapeDtypeStruct((NPAD, D), jnp.float32),
    )(P, hp1, dinv, W2t, b2r)


def _fin_body(q_ref, hp2_ref, dinv_ref, o_ref):
    o_ref[...] = jnp.tanh(
        dinv_ref[...] * (q_ref[0] + q_ref[1] + hp2_ref[...]))


def _fin_call(Q, hp2, dinv):
    return pl.pallas_call(
        _fin_body,
        grid=(NPAD // RB,),
        in_specs=[
            pl.BlockSpec((2, RB, D), lambda i: (0, i, 0)),
            pl.BlockSpec((RB, D), lambda i: (i, 0)),
            pl.BlockSpec((RB, 1), lambda i: (i, 0)),
        ],
        out_specs=pl.BlockSpec((RB, D), lambda i: (i, 0)),
        out_shape=jax.ShapeDtypeStruct((NPAD, D), jnp.float32),
    )(Q, hp2, dinv)


def kernel(x, edge_index, W1, b1, W2, b2):
    row = edge_index[0].astype(jnp.int32)
    col = edge_index[1].astype(jnp.int32)
    pad = EPAD - E
    fill = N + (jnp.arange(pad, dtype=jnp.int32) % (NPAD - N))
    rowf = jnp.concatenate([row, fill])
    colf = jnp.concatenate([col, fill])
    rowp = rowf.reshape(NCHUNK // 4, 4 * CHUNK)
    colp = colf.reshape(NCHUNK // 4, 4 * CHUNK)
    xp = jnp.zeros((NPAD, D), jnp.float32).at[:N].set(x)
    W1t = W1.T
    W2t = W2.T
    b1r = b1[None, :]
    b2r = b2[None, :]

    degp = _deg_kernel(rowp)
    hp1, dinv = _first_call(degp, xp, W1t, b1r)
    P = _edge_scatter(hp1, rowp, colp)
    hp2 = _mid_call(P, hp1, dinv, W2t, b2r)
    Q = _edge_scatter(hp2, rowp, colp)
    out = _fin_call(Q, hp2, dinv)
    return out[:N]


# speedup vs baseline: 1.1939x; 1.1939x over previous
import functools
import jax
import jax.numpy as jnp
from jax import lax
from jax.experimental import pallas as pl
from jax.experimental.pallas import tpu as pltpu
from jax.experimental.pallas import tpu_sc as plsc

N = 10000
E = 320000
D = 128
NC = 2
NS = 16
NW = NC * NS
CHUNK = 32
CPW = 320
NCHUNK = CPW * NW
EPAD = NCHUNK * CHUNK
NB = 8
SEG = 160
SEGP = SEG // 4
NPAD = 10240
RPS = NPAD // NS
DEGW = 16
RB = 1024

_mesh = plsc.VectorSubcoreMesh(
    core_axis_name="c", subcore_axis_name="s", num_cores=NC, num_subcores=NS)


@functools.partial(
    pl.kernel,
    out_type=jax.ShapeDtypeStruct((NC, NPAD, DEGW), jnp.float32),
    mesh=_mesh,
    scratch_types=[
        pltpu.VMEM((CPW // 4, 4 * CHUNK), jnp.int32),
        pltpu.VMEM((4 * CHUNK, DEGW), jnp.float32),
        pltpu.VMEM((16, DEGW), jnp.float32),
        pltpu.VMEM_SHARED((NPAD, DEGW), jnp.float32),
    ],
)
def _deg_kernel(row_hbm, out_hbm, idx_v, ones_v, zb_v, deg_sh):
    cid = lax.axis_index("c")
    sid = lax.axis_index("s")
    wid = sid * NC + cid

    def fill_ones(i, _):
        ones_v[i, :] = jnp.ones((DEGW,), jnp.float32)
        return 0
    lax.fori_loop(0, 4 * CHUNK, fill_ones, 0)

    def fill_z(i, _):
        zb_v[i, :] = jnp.zeros((DEGW,), jnp.float32)
        return 0
    lax.fori_loop(0, 16, fill_z, 0)

    def zero_acc(i, _):
        pltpu.sync_copy(zb_v, deg_sh.at[pl.ds(sid * RPS + i * 16, 16)])
        return 0
    lax.fori_loop(0, RPS // 16, zero_acc, 0)
    plsc.subcore_barrier()

    pltpu.sync_copy(row_hbm.at[pl.ds(wid * (CPW // 4), CPW // 4)], idx_v)

    def body(j, _):
        pltpu.sync_copy(ones_v, deg_sh.at[idx_v.at[j]], add=True)
        return 0
    lax.fori_loop(0, CPW // 4, body, 0)
    plsc.subcore_barrier()

    pltpu.sync_copy(deg_sh.at[pl.ds(sid * RPS, RPS)],
                    out_hbm.at[cid, pl.ds(sid * RPS, RPS)])


@functools.partial(
    pl.kernel,
    out_type=jax.ShapeDtypeStruct((NC, NPAD, D), jnp.float32),
    mesh=_mesh,
    scratch_types=[
        pltpu.VMEM((SEGP, 4 * CHUNK), jnp.int32),
        pltpu.VMEM((SEGP, 4 * CHUNK), jnp.int32),
        pltpu.VMEM((NB, CHUNK, D), jnp.float32),
        pltpu.VMEM((16, D), jnp.float32),
        pltpu.VMEM_SHARED((NPAD, D), jnp.float32),
        [pltpu.SemaphoreType.DMA] * NB,
        [pltpu.SemaphoreType.DMA] * NB,
    ],
)
def _edge_scatter(hp_hbm, row_hbm, col_hbm, out_hbm,
                  idxr_v, idxc_v, rows_v, zb_v, acc_sh, gsems, ssems):
    cid = lax.axis_index("c")
    sid = lax.axis_index("s")
    wid = sid * NC + cid

    def fill_z(i, _):
        def fz(k, _):
            zb_v[i, pl.ds(k * 16, 16)] = jnp.zeros((16,), jnp.float32)
            return 0
        lax.fori_loop(0, D // 16, fz, 0)
        return 0
    lax.fori_loop(0, 16, fill_z, 0)

    def zero_acc(i, _):
        pltpu.sync_copy(zb_v, acc_sh.at[pl.ds(sid * RPS + i * 16, 16)])
        return 0
    lax.fori_loop(0, RPS // 16, zero_acc, 0)
    plsc.subcore_barrier()

    GD = NB - 2
    for h in range(CPW // SEG):
        base = wid * (CPW // 4) + h * SEGP
        pltpu.sync_copy(row_hbm.at[pl.ds(base, SEGP)], idxr_v)
        pltpu.sync_copy(col_hbm.at[pl.ds(base, SEGP)], idxc_v)

        def idx_of(v, j):
            return v.at[j // 4, pl.ds((j % 4) * CHUNK, CHUNK)]

        def wait_g(b):
            pltpu.make_async_copy(
                hp_hbm.at[idx_of(idxr_v, 0)], rows_v.at[b], gsems[b]).wait()

        def start_g(b, j):
            pltpu.async_copy(
                hp_hbm.at[idx_of(idxr_v, j)], rows_v.at[b], gsems[b])

        def wait_s(b):
            pltpu.make_async_copy(
                rows_v.at[b], acc_sh.at[idx_of(idxc_v, 0)], ssems[b]).wait()

        def start_s(b, j):
            pltpu.async_copy(
                rows_v.at[b], acc_sh.at[idx_of(idxc_v, j)], ssems[b],
                add=True)

        for b in range(GD):
            start_g(b, b)

        for j in range(NB):
            b = j % NB
            wait_g(b)
            start_s(b, j)
            jd = j + GD
            if jd - NB >= 0:
                wait_s(jd % NB)
            start_g(jd % NB, jd)

        def group(g, _):
            for b in range(NB):
                j = g * NB + b
                wait_g(b)
                start_s(b, j)
                wait_s((b + GD) % NB)
                start_g((b + GD) % NB, j + GD)
            return 0
        lax.fori_loop(1, SEG // NB - 1, group, 0)

        for j in range(SEG - NB, SEG):
            b = j % NB
            wait_g(b)
            start_s(b, j)
            jd = j + GD
            if jd < SEG:
                wait_s(jd % NB)
                start_g(jd % NB, jd)

        for b in range(NB):
            wait_s(b)
    plsc.subcore_barrier()

    pltpu.sync_copy(acc_sh.at[pl.ds(sid * RPS, RPS)],
                    out_hbm.at[cid, pl.ds(sid * RPS, RPS)])


def _lin_body(x_ref, w_ref, b_ref, z_ref):
    z_ref[...] = jnp.dot(x_ref[...], w_ref[...],
                         preferred_element_type=jnp.float32) + b_ref[...]


def _lin_call(xp, W1t, b1r):
    return pl.pallas_call(
        _lin_body,
        grid=(NPAD // RB,),
        in_specs=[
            pl.BlockSpec((RB, D), lambda i: (i, 0)),
            pl.BlockSpec((D, D), lambda i: (0, 0)),
            pl.BlockSpec((1, D), lambda i: (0, 0)),
        ],
        out_specs=pl.BlockSpec((RB, D), lambda i: (i, 0)),
        out_shape=jax.ShapeDtypeStruct((NPAD, D), jnp.float32),
    )(xp, W1t, b1r)


def _first_body(deg_ref, z_ref, hp_ref, dinv_ref):
    deg = deg_ref[0, :, 0:1] + deg_ref[1, :, 0:1] + 1.0
    dinv = lax.rsqrt(deg)
    hp_ref[...] = dinv * z_ref[...]
    dinv_ref[...] = dinv


def _first_call(degp, z1):
    return pl.pallas_call(
        _first_body,
        grid=(NPAD // RB,),
        in_specs=[
            pl.BlockSpec((2, RB, DEGW), lambda i: (0, i, 0)),
            pl.BlockSpec((RB, D), lambda i: (i, 0)),
        ],
        out_specs=[
            pl.BlockSpec((RB, D), lambda i: (i, 0)),
            pl.BlockSpec((RB, 1), lambda i: (i, 0)),
        ],
        out_shape=[
            jax.ShapeDtypeStruct((NPAD, D), jnp.float32),
            jax.ShapeDtypeStruct((NPAD, 1), jnp.float32),
        ],
    )(degp, z1)


def _mid_body(p_ref, hp_ref, dinv_ref, w_ref, b_ref, hp2_ref):
    dinv = dinv_ref[...]
    h1 = jnp.tanh(dinv * (p_ref[0] + p_ref[1] + hp_ref[...]))
    z = jnp.dot(h1, w_ref[...], preferred_element_type=jnp.float32) + b_ref[...]
    hp2_ref[...] = dinv * z


def _mid_call(P, hp1, dinv, W2t, b2r):
    return pl.pallas_call(
        _mid_body,
        grid=(NPAD // RB,),
        in_specs=[
            pl.BlockSpec((2, RB, D), lambda i: (0, i, 0)),
            pl.BlockSpec((RB, D), lambda i: (i, 0)),
            pl.BlockSpec((RB, 1), lambda i: (i, 0)),
            pl.BlockSpec((D, D), lambda i: (0, 0)),
            pl.BlockSpec((1, D), lambda i: (0, 0)),
        ],
        out_specs=pl.BlockSpec((RB, D), lambda i: (i, 0)),
        out_shape=jax.ShapeDtypeStruct((NPAD, D), jnp.float32),
    )(P, hp1, dinv, W2t, b2r)


def _fin_body(q_ref, hp2_ref, dinv_ref, o_ref):
    o_ref[...] = jnp.tanh(
        dinv_ref[...] * (q_ref[0] + q_ref[1] + hp2_ref[...]))


def _fin_call(Q, hp2, dinv):
    return pl.pallas_call(
        _fin_body,
        grid=(NPAD // RB,),
        in_specs=[
            pl.BlockSpec((2, RB, D), lambda i: (0, i, 0)),
            pl.BlockSpec((RB, D), lambda i: (i, 0)),
            pl.BlockSpec((RB, 1), lambda i: (i, 0)),
        ],
        out_specs=pl.BlockSpec((RB, D), lambda i: (i, 0)),
        out_shape=jax.ShapeDtypeStruct((NPAD, D), jnp.float32),
    )(Q, hp2, dinv)


def kernel(x, edge_index, W1, b1, W2, b2):
    row = edge_index[0].astype(jnp.int32)
    col = edge_index[1].astype(jnp.int32)
    pad = EPAD - E
    fill = N + (jnp.arange(pad, dtype=jnp.int32) % (NPAD - N))
    rowf = jnp.concatenate([row, fill])
    colf = jnp.concatenate([col, fill])
    rowp = rowf.reshape(NCHUNK // 4, 4 * CHUNK)
    colp = colf.reshape(NCHUNK // 4, 4 * CHUNK)
    xp = jnp.zeros((NPAD, D), jnp.float32).at[:N].set(x)
    W1t = W1.T
    W2t = W2.T
    b1r = b1[None, :]
    b2r = b2[None, :]

    z1 = _lin_call(xp, W1t, b1r)
    degp = _deg_kernel(rowp)
    hp1, dinv = _first_call(degp, z1)
    P = _edge_scatter(hp1, rowp, colp)
    hp2 = _mid_call(P, hp1, dinv, W2t, b2r)
    Q = _edge_scatter(hp2, rowp, colp)
    out = _fin_call(Q, hp2, dinv)
    return out[:N]
